# baseline (speedup 1.0000x reference)
import functools

import jax
import jax.numpy as jnp
from jax import lax
from jax.experimental import pallas as pl
from jax.experimental.pallas import tpu as pltpu
from jax.experimental.pallas import tpu_sc as plsc

B, S, MK, MF, D = 16, 512, 8, 16, 128
LK, LF = 1024, 2048
TBS = B * S
TB_SRC = TBS * (MK + 1)
IMG_SRC = B * (LK + 1)
IMG_BASE = TB_SRC
NLP_BASE = TB_SRC + IMG_SRC
MASK_ROW = NLP_BASE + IMG_SRC

OUT_T = TBS * (MF + 1)
OUT_I0 = OUT_T
IN_ROWS = 2 * B * (LF + 1)
OUT_ROWS = OUT_T + IN_ROWS
U_LEN = LF + 1

NW = 32
G_PER_W = TBS // NW
TR_PER_W = G_PER_W * (MF + 1)
TCH = TR_PER_W // 128
W_ROWS = 2048
UCH = W_ROWS // 128
TAIL = IN_ROWS - NW * W_ROWS


def _sc_revert(table, tidx, iidx, nidx):
    mesh = plsc.VectorSubcoreMesh(core_axis_name="c", subcore_axis_name="s")

    @functools.partial(
        pl.kernel,
        mesh=mesh,
        out_type=jax.ShapeDtypeStruct((OUT_ROWS, D), jnp.float32),
        compiler_params=pltpu.CompilerParams(needs_layout_passes=False),
        scratch_types=[
            pltpu.VMEM((G_PER_W * MF,), jnp.int32),
            pltpu.VMEM((TCH, 128), jnp.int32),
            pltpu.VMEM((2 * LF,), jnp.int32),
            pltpu.VMEM((UCH, 128), jnp.int32),
            pltpu.VMEM((TAIL,), jnp.int32),
            pltpu.VMEM((128, D), jnp.float32),
            pltpu.SemaphoreType.DMA,
        ],
    )
    def k(table_h, tidx_h, iidx_h, nidx_h, out_h,
          tidx_v, gidx_t, idx2, gidx_u, gidx_x, rows_v, sem):
        w = lax.axis_index("s") * 2 + lax.axis_index("c")
        lane = lax.iota(jnp.int32, 16)

        pltpu.sync_copy(tidx_h.at[pl.ds(w * (G_PER_W * MF), G_PER_W * MF)], tidx_v)

        def fill_t(r, carry):
            for jj in range(8):
                lr = r * 128 + jj * 16 + lane
                lbs = lax.div(lr, jnp.int32(17))
                m = lr - lbs * 17
                pos = jnp.maximum(lbs * MF + m - 1, 0)
                e = plsc.load_gather(tidx_v, [pos])
                base = (w * G_PER_W + lbs) * (MK + 1)
                gi = jnp.where(m == 0, base,
                               jnp.where(e < MK, base + 1 + e, MASK_ROW))
                gidx_t[r, pl.ds(jj * 16, 16)] = gi
            return carry

        lax.fori_loop(0, TCH, fill_t, 0)

        def gath_t(c, carry):
            pltpu.async_copy(table_h.at[gidx_t.at[c]], rows_v, sem).wait()
            pltpu.sync_copy(rows_v, out_h.at[pl.ds(w * TR_PER_W + c * 128, 128)])
            return carry

        lax.fori_loop(0, TCH, gath_t, 0)

        r0 = w * W_ROWS
        u0 = r0 // U_LEN
        units = [u0, jnp.minimum(u0 + 1, 2 * B - 1)]
        for t in range(2):
            uu = units[t]
            bb = lax.rem(uu, B)

            @pl.when(uu < B)
            def _():
                pltpu.sync_copy(iidx_h.at[pl.ds(bb * LF, LF)],
                                idx2.at[pl.ds(t * LF, LF)])

            @pl.when(uu >= B)
            def _():
                pltpu.sync_copy(nidx_h.at[pl.ds(bb * LF, LF)],
                                idx2.at[pl.ds(t * LF, LF)])

        def unit_gi(R):
            u = lax.div(R, jnp.int32(U_LEN))
            p = R - u * U_LEN
            tb_vec = jnp.where(u < B, IMG_BASE + u * (LK + 1),
                               NLP_BASE + (u - B) * (LK + 1))
            pos = (u - u0) * LF + jnp.maximum(p - 1, 0)
            e = plsc.load_gather(idx2, [pos])
            return jnp.where(p == 0, tb_vec,
                             jnp.where(e < LK, tb_vec + 1 + e, MASK_ROW))

        def fill_u(r, carry):
            for jj in range(8):
                R = r0 + r * 128 + jj * 16 + lane
                gidx_u[r, pl.ds(jj * 16, 16)] = unit_gi(R)
            return carry

        lax.fori_loop(0, UCH, fill_u, 0)

        def gath_u(c, carry):
            pltpu.async_copy(table_h.at[gidx_u.at[c]], rows_v, sem).wait()
            pltpu.sync_copy(rows_v, out_h.at[pl.ds(OUT_I0 + r0 + c * 128, 128)])
            return carry

        lax.fori_loop(0, UCH, gath_u, 0)

        @pl.when(w == NW - 1)
        def _():
            for jj in range(TAIL // 16):
                R = NW * W_ROWS + jj * 16 + lane
                gidx_x[pl.ds(jj * 16, 16)] = unit_gi(R)
            pltpu.async_copy(table_h.at[gidx_x], rows_v.at[pl.ds(0, TAIL)],
                             sem).wait()
            pltpu.sync_copy(rows_v.at[pl.ds(0, TAIL)],
                            out_h.at[pl.ds(OUT_I0 + NW * W_ROWS, TAIL)])

    return k(table, tidx, iidx, nidx)


def kernel(temporal_block_encoding, img, nlp, mask_token,
           temporal_block_revert_idx, img_revert_idx, nlp_revert_idx):
    table = jnp.concatenate(
        [
            temporal_block_encoding.reshape(TB_SRC, D),
            img.reshape(IMG_SRC, D),
            nlp.reshape(IMG_SRC, D),
            mask_token.astype(jnp.float32),
        ],
        axis=0,
    )
    out = _sc_revert(
        table,
        temporal_block_revert_idx.reshape(-1).astype(jnp.int32),
        img_revert_idx.reshape(-1).astype(jnp.int32),
        nlp_revert_idx.reshape(-1).astype(jnp.int32),
    )
    return out.reshape(-1)
